# baseline (speedup 1.0000x reference)
import functools
import math

import jax
import jax.numpy as jnp
from jax import lax
from jax.experimental import pallas as pl
from jax.experimental.pallas import tpu as pltpu
from jax.experimental.pallas import tpu_sc as plsc

VOCAB = 100000
EMBED = 128
BATCH = 4096
SEQ = 50

ROWS = BATCH * SEQ
NC, NS = 2, 16
NW = NC * NS
PER_W = ROWS // NW
C = 128
NCH = PER_W // C
NBUF = 2
LANES = 16
VECS_PER_ROW = EMBED // LANES

SCALE = math.sqrt(float(EMBED))

_mesh = plsc.VectorSubcoreMesh(core_axis_name="c", subcore_axis_name="s")


@functools.partial(
    pl.kernel,
    mesh=_mesh,
    out_type=jax.ShapeDtypeStruct((ROWS, EMBED), jnp.float32),
    scratch_types=[
        pltpu.VMEM((NCH, C), jnp.int32),
        pltpu.VMEM((NBUF, C, EMBED), jnp.float32),
        pltpu.VMEM((NBUF, C, EMBED), jnp.float32),
        pltpu.SemaphoreType.DMA,
        pltpu.SemaphoreType.DMA,
        pltpu.SemaphoreType.DMA,
        pltpu.SemaphoreType.DMA,
    ],
)
def _embed_lookup(table_hbm, x_hbm, out_hbm, idx_v, gbuf, sbuf,
                  gsem0, gsem1, ssem0, ssem1):
    wid = lax.axis_index("s") * NC + lax.axis_index("c")
    base = wid * PER_W
    gsems = [gsem0, gsem1]
    ssems = [ssem0, ssem1]

    pltpu.sync_copy(x_hbm.at[wid], idx_v)

    def gather_start(j, b):
        pltpu.async_copy(table_hbm.at[idx_v.at[j]], gbuf.at[b], gsems[b])

    def gather_wait(b):
        pltpu.make_async_copy(out_hbm.at[pl.ds(0, C)], gbuf.at[b],
                              gsems[b]).wait()

    def scatter_start(j, b):
        pltpu.async_copy(sbuf.at[b], out_hbm.at[pl.ds(base + j * C, C)],
                         ssems[b])

    def scatter_wait(b):
        pltpu.make_async_copy(out_hbm.at[pl.ds(0, C)], sbuf.at[b],
                              ssems[b]).wait()

    def scale(b):
        gb = gbuf.at[b]
        sb = sbuf.at[b]

        def row_body(r, c2):
            for k in range(VECS_PER_ROW):
                sl = pl.ds(k * LANES, LANES)
                sb[r, sl] = gb[r, sl] * SCALE
            return c2

        lax.fori_loop(0, C, row_body, 0, unroll=4)

    for b in range(NBUF):
        gather_start(b, b)

    for b in range(NBUF):
        gather_wait(b)
        scale(b)
        scatter_start(b, b)
        gather_start(b + NBUF, b)

    def group_body(g, carry):
        for b in range(NBUF):
            j = g * NBUF + b
            gather_wait(b)
            scatter_wait(b)
            scale(b)
            scatter_start(j, b)
            gather_start(j + NBUF, b)
        return carry

    lax.fori_loop(1, NCH // NBUF - 1, group_body, 0)

    for b in range(NBUF):
        j = NCH - NBUF + b
        gather_wait(b)
        scatter_wait(b)
        scale(b)
        scatter_start(j, b)

    for b in range(NBUF):
        scatter_wait(b)


def kernel(x, table):
    xr = x.astype(jnp.int32).reshape(NW, NCH, C)
    out = _embed_lookup(table, xr)
    return out.reshape(BATCH, SEQ, EMBED)


# speedup vs baseline: 1.3796x; 1.3796x over previous
import functools
import math

import jax
import jax.numpy as jnp
from jax import lax
from jax.experimental import pallas as pl
from jax.experimental.pallas import tpu as pltpu
from jax.experimental.pallas import tpu_sc as plsc

VOCAB = 100000
EMBED = 128
BATCH = 4096
SEQ = 50

NC, NS = 2, 16
NW = NC * NS
B_PER_W = BATCH // NW
BPC = 2
CR = BPC * SEQ
NCH = B_PER_W // BPC
NBUF = 2
LANES = 16
VECS_PER_ROW = EMBED // LANES

SCALE = math.sqrt(float(EMBED))

_mesh = plsc.VectorSubcoreMesh(core_axis_name="c", subcore_axis_name="s")


@functools.partial(
    pl.kernel,
    mesh=_mesh,
    out_type=jax.ShapeDtypeStruct((BATCH, SEQ, EMBED), jnp.float32),
    compiler_params=pltpu.CompilerParams(use_tc_tiling_on_sc=True),
    scratch_types=[
        pltpu.VMEM((NCH, CR), jnp.int32),
        pltpu.VMEM((NBUF, CR, EMBED), jnp.float32),
        pltpu.VMEM((NBUF, BPC, SEQ, EMBED), jnp.float32),
        pltpu.SemaphoreType.DMA,
        pltpu.SemaphoreType.DMA,
        pltpu.SemaphoreType.DMA,
        pltpu.SemaphoreType.DMA,
    ],
)
def _embed_lookup(table_hbm, x_hbm, out_hbm, idx_v, gbuf, sbuf,
                  gsem0, gsem1, ssem0, ssem1):
    wid = lax.axis_index("s") * NC + lax.axis_index("c")
    batch0 = wid * B_PER_W
    gsems = [gsem0, gsem1]
    ssems = [ssem0, ssem1]

    pltpu.sync_copy(x_hbm.at[wid], idx_v)

    def gather_start(j, b):
        pltpu.async_copy(table_hbm.at[idx_v.at[j]], gbuf.at[b], gsems[b])

    def gather_wait(b):
        pltpu.make_async_copy(table_hbm.at[idx_v.at[0]], gbuf.at[b],
                              gsems[b]).wait()

    def scatter_start(j, b):
        pltpu.async_copy(sbuf.at[b],
                         out_hbm.at[pl.ds(batch0 + j * BPC, BPC)],
                         ssems[b])

    def scatter_wait(b):
        pltpu.make_async_copy(sbuf.at[b], out_hbm.at[pl.ds(batch0, BPC)],
                              ssems[b]).wait()

    def scale(b):
        gb = gbuf.at[b]
        sb = sbuf.at[b]
        for d in range(BPC):
            def row_body(r, c2, _d=d):
                for k in range(VECS_PER_ROW):
                    sl = pl.ds(k * LANES, LANES)
                    sb[_d, r, sl] = gb[_d * SEQ + r, sl] * SCALE
                return c2

            lax.fori_loop(0, SEQ, row_body, 0, unroll=2)

    for b in range(NBUF):
        gather_start(b, b)

    for b in range(NBUF):
        gather_wait(b)
        scale(b)
        scatter_start(b, b)
        gather_start(b + NBUF, b)

    def group_body(g, carry):
        for b in range(NBUF):
            j = g * NBUF + b
            gather_wait(b)
            scatter_wait(b)
            scale(b)
            scatter_start(j, b)
            gather_start(j + NBUF, b)
        return carry

    lax.fori_loop(1, NCH // NBUF - 1, group_body, 0)

    for b in range(NBUF):
        j = NCH - NBUF + b
        gather_wait(b)
        scatter_wait(b)
        scale(b)
        scatter_start(j, b)

    for b in range(NBUF):
        scatter_wait(b)


def kernel(x, table):
    xr = x.astype(jnp.int32).reshape(NW, NCH, CR)
    return _embed_lookup(table, xr)
